# Initial kernel scaffold; baseline (speedup 1.0000x reference)
import jax
import jax.numpy as jnp
from jax.experimental import pallas as pl


def kernel(x, edge_index, W1, b1, W2, b2, Wx, bx, Wh, bh):
    raise NotImplementedError("write your pallas kernel here")


# speedup vs baseline: 1.6757x; 1.6757x over previous
import functools

import jax
import jax.numpy as jnp
from jax import lax
from jax.experimental import pallas as pl
from jax.experimental.pallas import tpu as pltpu
from jax.experimental.pallas import tpu_sc as plsc

_NC = 2
_NS = 16
_NW = _NC * _NS
_CHUNK = 128
_LANES = 16


def _rup(v, m):
    return -(-v // m) * m


@functools.lru_cache(maxsize=None)
def _make_partition(wch, half):
    lch = wch + 1
    lcap = lch * _CHUNK
    mesh = plsc.VectorSubcoreMesh(core_axis_name="c", subcore_axis_name="s")

    @functools.partial(
        pl.kernel,
        out_type=[
            jax.ShapeDtypeStruct((_NW, 2, lch, _CHUNK), jnp.int32),
            jax.ShapeDtypeStruct((_NW, 2, lch, _CHUNK), jnp.int32),
            jax.ShapeDtypeStruct((_NW, _CHUNK), jnp.int32),
        ],
        mesh=mesh,
        scratch_types=[
            pltpu.VMEM((wch, _CHUNK), jnp.int32),
            pltpu.VMEM((wch, _CHUNK), jnp.int32),
            pltpu.VMEM((lcap + _LANES,), jnp.int32),
            pltpu.VMEM((lcap + _LANES,), jnp.int32),
            pltpu.VMEM((lcap + _LANES,), jnp.int32),
            pltpu.VMEM((lcap + _LANES,), jnp.int32),
            pltpu.VMEM((lch, _CHUNK), jnp.int32),
            pltpu.VMEM((_LANES,), jnp.int32),
        ],
        compiler_params=pltpu.CompilerParams(needs_layout_passes=False),
    )
    def partition(src_hbm, dst_hbm, srcl_hbm, dstl_hbm, cnt_hbm,
                  src_in, dst_in, lsrc, ldst, hsrc, hdst, d2, cntv):
        c = lax.axis_index("c")
        s = lax.axis_index("s")
        w = s * _NC + c
        pltpu.sync_copy(src_hbm.at[w], src_in)
        pltpu.sync_copy(dst_hbm.at[w], dst_in)

        lane = lax.iota(jnp.int32, _LANES)
        trash = lane + lcap

        def chunk_body(j, carry):
            cl, ch = carry
            for k in range(_CHUNK // _LANES):
                sl = pl.ds(k * _LANES, _LANES)
                dv = dst_in[j, sl]
                sv = src_in[j, sl]
                m_lo = dv < half
                ones = jnp.where(m_lo, jnp.int32(1), jnp.int32(0))
                idx_lo = trash
                idx_hi = trash
                for q in range(_LANES):
                    oq = ones[q]
                    tl = jnp.where(oq == 1, cl, lcap + q)
                    th = jnp.where(oq == 1, lcap + q, ch)
                    idx_lo = jnp.where(lane == q, tl, idx_lo)
                    idx_hi = jnp.where(lane == q, th, idx_hi)
                    cl = cl + oq
                    ch = ch + (1 - oq)
                plsc.store_scatter(lsrc, [idx_lo], sv)
                plsc.store_scatter(ldst, [idx_lo], dv)
                plsc.store_scatter(hsrc, [idx_hi], sv)
                plsc.store_scatter(hdst, [idx_hi], dv - half)
            return cl, ch

        cl, ch = lax.fori_loop(0, wch, chunk_body, (jnp.int32(0), jnp.int32(0)))

        zeros16 = jnp.zeros((_LANES,), jnp.int32)
        dummy16 = jnp.full((_LANES,), half, jnp.int32)
        for k in range(_CHUNK // _LANES):
            off = k * _LANES
            lsrc[pl.ds(cl + off, _LANES)] = zeros16
            ldst[pl.ds(cl + off, _LANES)] = dummy16
            hsrc[pl.ds(ch + off, _LANES)] = zeros16
            hdst[pl.ds(ch + off, _LANES)] = dummy16

        groups = _CHUNK // _LANES

        def emit(flat_ref, out_view):
            def body(q, carry):
                d2[q // groups, pl.ds((q % groups) * _LANES, _LANES)] = (
                    flat_ref[pl.ds(q * _LANES, _LANES)])
                return carry
            lax.fori_loop(0, lch * groups, body, 0)
            pltpu.sync_copy(d2, out_view)

        emit(lsrc, srcl_hbm.at[w].at[0])
        emit(ldst, dstl_hbm.at[w].at[0])
        emit(hsrc, srcl_hbm.at[w].at[1])
        emit(hdst, dstl_hbm.at[w].at[1])

        nch_lo = (cl + _CHUNK - 1) >> 7
        nch_hi = (ch + _CHUNK - 1) >> 7
        lane = lax.iota(jnp.int32, _LANES)
        cvec = jnp.where(lane == 0, nch_lo,
                         jnp.where(lane == 1, nch_hi, 0))
        cntv[...] = cvec
        pltpu.sync_copy(cntv, cnt_hbm.at[w].at[pl.ds(0, _LANES)])

    return partition


@functools.lru_cache(maxsize=None)
def _make_segsum(d, wch, half, acc_rows, slabz, wb, out_rows):
    lch = wch + 1
    mesh = plsc.VectorSubcoreMesh(core_axis_name="c", subcore_axis_name="s")

    @functools.partial(
        pl.kernel,
        out_type=jax.ShapeDtypeStruct((out_rows, d), jnp.float32),
        mesh=mesh,
        scratch_types=[
            pltpu.VMEM((2, lch, _CHUNK), jnp.int32),
            pltpu.VMEM((2, lch, _CHUNK), jnp.int32),
            pltpu.VMEM((2, _CHUNK), jnp.int32),
            pltpu.VMEM((_CHUNK, d), jnp.float32),
            pltpu.VMEM_SHARED((acc_rows, d), jnp.float32),
            pltpu.SemaphoreType.DMA,
        ],
        compiler_params=pltpu.CompilerParams(needs_layout_passes=False),
    )
    def segsum(sup_hbm, srcl_hbm, dstl_hbm, cnt_hbm, zero_hbm, out_hbm,
               sidx, didx, cnts, rows, acc, sem):
        c = lax.axis_index("c")
        s = lax.axis_index("s")
        pltpu.sync_copy(zero_hbm, acc.at[pl.ds(s * slabz, slabz)])
        pltpu.sync_copy(srcl_hbm.at[2 * s].at[c], sidx.at[0])
        pltpu.sync_copy(srcl_hbm.at[2 * s + 1].at[c], sidx.at[1])
        pltpu.sync_copy(dstl_hbm.at[2 * s].at[c], didx.at[0])
        pltpu.sync_copy(dstl_hbm.at[2 * s + 1].at[c], didx.at[1])
        pltpu.sync_copy(cnt_hbm.at[pl.ds(2 * s, 2)], cnts)
        plsc.subcore_barrier()

        for i in range(2):
            cv = cnts[i, pl.ds(0, _LANES)]
            nch = jnp.where(c == 0, cv[0], cv[1])

            def body(j, carry, i=i):
                pltpu.async_copy(sup_hbm.at[sidx.at[i].at[j]], rows,
                                 sem).wait()
                pltpu.sync_copy(rows, acc.at[didx.at[i].at[j]], add=True)
                return carry

            lax.fori_loop(0, nch, body, 0)

        plsc.subcore_barrier()
        pltpu.sync_copy(acc.at[pl.ds(s * wb, wb)],
                        out_hbm.at[pl.ds(c * half + s * wb, wb)])

    return segsum


_BLK = 1000


def _mm_bias_body(x_ref, w_ref, b_ref, o_ref):
    o_ref[...] = (jnp.dot(x_ref[...], w_ref[...],
                          preferred_element_type=jnp.float32) + b_ref[...])


def _mm_bias(x2, w, b):
    n, k = x2.shape
    m = w.shape[1]
    return pl.pallas_call(
        _mm_bias_body,
        grid=(n // _BLK,),
        in_specs=[
            pl.BlockSpec((_BLK, k), lambda i: (i, 0)),
            pl.BlockSpec((k, m), lambda i: (0, 0)),
            pl.BlockSpec((1, m), lambda i: (0, 0)),
        ],
        out_specs=pl.BlockSpec((_BLK, m), lambda i: (i, 0)),
        out_shape=jax.ShapeDtypeStruct((n, m), jnp.float32),
    )(x2, w, b.reshape(1, m))


def _combine_body(agg_ref, res_ref, w2_ref, b2_ref, wn_ref, bn_ref,
                  out_ref, supn_ref):
    agg = agg_ref[...]
    r = (jnp.dot(res_ref[...], w2_ref[...],
                 preferred_element_type=jnp.float32) + b2_ref[...])
    g = jax.nn.sigmoid(agg + r)
    o = g * jnp.tanh(agg) + (1.0 - g) * r
    out_ref[...] = o
    supn_ref[...] = (jnp.dot(o, wn_ref[...],
                             preferred_element_type=jnp.float32) + bn_ref[...])


def _combine(agg, res, w2, b2, wn, bn):
    n, d = res.shape
    m = wn.shape[1]
    return pl.pallas_call(
        _combine_body,
        grid=(n // _BLK,),
        in_specs=[
            pl.BlockSpec((_BLK, d), lambda i: (i, 0)),
            pl.BlockSpec((_BLK, d), lambda i: (i, 0)),
            pl.BlockSpec((d, d), lambda i: (0, 0)),
            pl.BlockSpec((1, d), lambda i: (0, 0)),
            pl.BlockSpec((d, m), lambda i: (0, 0)),
            pl.BlockSpec((1, m), lambda i: (0, 0)),
        ],
        out_specs=[
            pl.BlockSpec((_BLK, d), lambda i: (i, 0)),
            pl.BlockSpec((_BLK, m), lambda i: (i, 0)),
        ],
        out_shape=[
            jax.ShapeDtypeStruct((n, d), jnp.float32),
            jax.ShapeDtypeStruct((n, m), jnp.float32),
        ],
    )(agg, res, w2, b2.reshape(1, d), wn, bn.reshape(1, m))


def _gru_body(gx_ref, h_ref, wh_ref, bh_ref, o_ref):
    d = h_ref.shape[1]
    gh = (jnp.dot(h_ref[...], wh_ref[...],
                  preferred_element_type=jnp.float32) + bh_ref[...])
    gx = gx_ref[...]
    i_r, i_i, i_n = gx[:, :d], gx[:, d:2 * d], gx[:, 2 * d:]
    h_r, h_i, h_n = gh[:, :d], gh[:, d:2 * d], gh[:, 2 * d:]
    rg = jax.nn.sigmoid(i_r + h_r)
    ig = jax.nn.sigmoid(i_i + h_i)
    ng = jnp.tanh(i_n + rg * h_n)
    o_ref[...] = ng + ig * (h_ref[...] - ng)


def _gru(gx, h, wh, bh):
    n, d = h.shape
    return pl.pallas_call(
        _gru_body,
        grid=(n // _BLK,),
        in_specs=[
            pl.BlockSpec((_BLK, 3 * d), lambda i: (i, 0)),
            pl.BlockSpec((_BLK, d), lambda i: (i, 0)),
            pl.BlockSpec((d, 3 * d), lambda i: (0, 0)),
            pl.BlockSpec((1, 3 * d), lambda i: (0, 0)),
        ],
        out_specs=pl.BlockSpec((_BLK, d), lambda i: (i, 0)),
        out_shape=jax.ShapeDtypeStruct((n, d), jnp.float32),
    )(gx, h, wh, bh.reshape(1, 3 * d))


def kernel(x, edge_index, W1, b1, W2, b2, Wx, bx, Wh, bh):
    t_steps, n_nodes, d = x.shape
    layers = W1.shape[1]
    n_edges = edge_index.shape[1]

    wb = _rup(-(-n_nodes // _NW), 8)
    out_rows = wb * _NW
    half = wb * _NS
    slabz = _rup(-(-(half + 8) // _NS), 8)
    acc_rows = slabz * _NS

    per_worker_unit = _CHUNK * _NW
    epad = _rup(n_edges, per_worker_unit)
    wch = epad // per_worker_unit

    pad = epad - n_edges
    srcp = jnp.concatenate(
        [edge_index[0], jnp.zeros((pad,), jnp.int32)]).reshape(_NW, wch, _CHUNK)
    dstp = jnp.concatenate(
        [edge_index[1], jnp.full((pad,), 2 * half, jnp.int32)]
    ).reshape(_NW, wch, _CHUNK)
    zero_blk = jnp.zeros((slabz, d), jnp.float32)

    part = _make_partition(wch, half)
    srcl, dstl, cnts = part(srcp, dstp)
    segsum = _make_segsum(d, wch, half, acc_rows, slabz, wb, out_rows)

    steps = t_steps * layers
    sup0 = jnp.stack([_mm_bias(x[t], W1[t, 0], b1[t, 0])
                      for t in range(t_steps)])

    w2s, b2s, wns, bns, sup0s, res0s, l0f, lastf = [], [], [], [], [], [], [], []
    znd = jnp.zeros((n_nodes, d), jnp.float32)
    for t in range(t_steps):
        for l in range(layers):
            w2s.append(W2[t, l])
            b2s.append(b2[t, l])
            if l + 1 < layers:
                wns.append(jnp.pad(W1[t, l + 1], ((0, 0), (0, 2 * d))))
                bns.append(jnp.pad(b1[t, l + 1], (0, 2 * d)))
            else:
                wns.append(Wx)
                bns.append(bx)
            sup0s.append(sup0[t] if l == 0 else znd)
            res0s.append(x[t] if l == 0 else znd)
            l0f.append(l == 0)
            lastf.append(l == layers - 1)
    xs = (jnp.stack(w2s), jnp.stack(b2s), jnp.stack(wns), jnp.stack(bns),
          jnp.stack(sup0s), jnp.stack(res0s),
          jnp.array(l0f), jnp.array(lastf))

    def step(carry, xso):
        res, supw, h = carry
        w2i, b2i, wni, bni, sup0i, res0i, is_l0, is_last = xso
        sup_in = jnp.where(is_l0, sup0i, supw[:, :d])
        res_in = jnp.where(is_l0, res0i, res)
        agg = segsum(sup_in, srcl, dstl, cnts, zero_blk)
        res_out, supw_out = _combine(agg, res_in, w2i, b2i, wni, bni)
        h_new = _gru(supw_out, h, Wh, bh)
        h_out = jnp.where(is_last, h_new, h)
        return (res_out, supw_out, h_out), h_out

    init = (znd, jnp.zeros((n_nodes, 3 * d), jnp.float32), znd)
    _, hs = lax.scan(step, init, xs)
    return hs[layers - 1::layers]


# speedup vs baseline: 1.9090x; 1.1392x over previous
import functools

import jax
import jax.numpy as jnp
from jax import lax
from jax.experimental import pallas as pl
from jax.experimental.pallas import tpu as pltpu
from jax.experimental.pallas import tpu_sc as plsc

_NC = 2
_NS = 16
_NW = _NC * _NS
_CHUNK = 128
_LANES = 16


def _rup(v, m):
    return -(-v // m) * m


@functools.lru_cache(maxsize=None)
def _make_partition(wch, half):
    lch = wch + 1
    lcap = lch * _CHUNK
    mesh = plsc.VectorSubcoreMesh(core_axis_name="c", subcore_axis_name="s")

    @functools.partial(
        pl.kernel,
        out_type=[
            jax.ShapeDtypeStruct((_NW, 2, lch, _CHUNK), jnp.int32),
            jax.ShapeDtypeStruct((_NW, 2, lch, _CHUNK), jnp.int32),
            jax.ShapeDtypeStruct((_NW, _CHUNK), jnp.int32),
        ],
        mesh=mesh,
        scratch_types=[
            pltpu.VMEM((wch, _CHUNK), jnp.int32),
            pltpu.VMEM((wch, _CHUNK), jnp.int32),
            pltpu.VMEM((lcap + _LANES,), jnp.int32),
            pltpu.VMEM((lcap + _LANES,), jnp.int32),
            pltpu.VMEM((lcap + _LANES,), jnp.int32),
            pltpu.VMEM((lcap + _LANES,), jnp.int32),
            pltpu.VMEM((lch, _CHUNK), jnp.int32),
            pltpu.VMEM((_LANES,), jnp.int32),
        ],
        compiler_params=pltpu.CompilerParams(needs_layout_passes=False),
    )
    def partition(src_hbm, dst_hbm, srcl_hbm, dstl_hbm, cnt_hbm,
                  src_in, dst_in, lsrc, ldst, hsrc, hdst, d2, cntv):
        c = lax.axis_index("c")
        s = lax.axis_index("s")
        w = s * _NC + c
        pltpu.sync_copy(src_hbm.at[w], src_in)
        pltpu.sync_copy(dst_hbm.at[w], dst_in)

        lane = lax.iota(jnp.int32, _LANES)
        trash = lane + lcap

        def chunk_body(j, carry):
            cl, ch = carry
            for k in range(_CHUNK // _LANES):
                sl = pl.ds(k * _LANES, _LANES)
                dv = dst_in[j, sl]
                sv = src_in[j, sl]
                m_lo = dv < half
                ones = jnp.where(m_lo, jnp.int32(1), jnp.int32(0))
                idx_lo = trash
                idx_hi = trash
                for q in range(_LANES):
                    oq = ones[q]
                    tl = jnp.where(oq == 1, cl, lcap + q)
                    th = jnp.where(oq == 1, lcap + q, ch)
                    idx_lo = jnp.where(lane == q, tl, idx_lo)
                    idx_hi = jnp.where(lane == q, th, idx_hi)
                    cl = cl + oq
                    ch = ch + (1 - oq)
                plsc.store_scatter(lsrc, [idx_lo], sv)
                plsc.store_scatter(ldst, [idx_lo], dv)
                plsc.store_scatter(hsrc, [idx_hi], sv)
                plsc.store_scatter(hdst, [idx_hi], dv - half)
            return cl, ch

        cl, ch = lax.fori_loop(0, wch, chunk_body, (jnp.int32(0), jnp.int32(0)))

        zeros16 = jnp.zeros((_LANES,), jnp.int32)
        dummy16 = jnp.full((_LANES,), half, jnp.int32)
        for k in range(_CHUNK // _LANES):
            off = k * _LANES
            lsrc[pl.ds(cl + off, _LANES)] = zeros16
            ldst[pl.ds(cl + off, _LANES)] = dummy16
            hsrc[pl.ds(ch + off, _LANES)] = zeros16
            hdst[pl.ds(ch + off, _LANES)] = dummy16

        groups = _CHUNK // _LANES

        def emit(flat_ref, out_view):
            def body(q, carry):
                d2[q // groups, pl.ds((q % groups) * _LANES, _LANES)] = (
                    flat_ref[pl.ds(q * _LANES, _LANES)])
                return carry
            lax.fori_loop(0, lch * groups, body, 0)
            pltpu.sync_copy(d2, out_view)

        emit(lsrc, srcl_hbm.at[w].at[0])
        emit(ldst, dstl_hbm.at[w].at[0])
        emit(hsrc, srcl_hbm.at[w].at[1])
        emit(hdst, dstl_hbm.at[w].at[1])

        nch_lo = (cl + _CHUNK - 1) >> 7
        nch_hi = (ch + _CHUNK - 1) >> 7
        lane = lax.iota(jnp.int32, _LANES)
        cvec = jnp.where(lane == 0, nch_lo,
                         jnp.where(lane == 1, nch_hi, 0))
        cntv[...] = cvec
        pltpu.sync_copy(cntv, cnt_hbm.at[w].at[pl.ds(0, _LANES)])

    return partition


@functools.lru_cache(maxsize=None)
def _make_segsum(d, wch, half, acc_rows, slabz, wb, out_rows):
    lch = wch + 1
    mesh = plsc.VectorSubcoreMesh(core_axis_name="c", subcore_axis_name="s")

    @functools.partial(
        pl.kernel,
        out_type=jax.ShapeDtypeStruct((out_rows, d), jnp.float32),
        mesh=mesh,
        scratch_types=[
            pltpu.VMEM((2, lch, _CHUNK), jnp.int32),
            pltpu.VMEM((2, lch, _CHUNK), jnp.int32),
            pltpu.VMEM((2, _CHUNK), jnp.int32),
            pltpu.VMEM((_CHUNK, d), jnp.float32),
            pltpu.VMEM((_CHUNK, d), jnp.float32),
            pltpu.VMEM_SHARED((acc_rows, d), jnp.float32),
            pltpu.SemaphoreType.DMA,
            pltpu.SemaphoreType.DMA,
        ],
        compiler_params=pltpu.CompilerParams(needs_layout_passes=False),
    )
    def segsum(sup_hbm, srcl_hbm, dstl_hbm, cnt_hbm, zero_hbm, out_hbm,
               sidx, didx, cnts, rows_a, rows_b, acc, sem_a, sem_b):
        c = lax.axis_index("c")
        s = lax.axis_index("s")
        pltpu.sync_copy(zero_hbm, acc.at[pl.ds(s * slabz, slabz)])
        pltpu.sync_copy(srcl_hbm.at[2 * s].at[c], sidx.at[0])
        pltpu.sync_copy(srcl_hbm.at[2 * s + 1].at[c], sidx.at[1])
        pltpu.sync_copy(dstl_hbm.at[2 * s].at[c], didx.at[0])
        pltpu.sync_copy(dstl_hbm.at[2 * s + 1].at[c], didx.at[1])
        pltpu.sync_copy(cnt_hbm.at[pl.ds(2 * s, 2)], cnts)
        plsc.subcore_barrier()

        for i in range(2):
            cv = cnts[i, pl.ds(0, _LANES)]
            nch = jnp.where(c == 0, cv[0], cv[1])

            def gather(j, buf, sem, i=i):
                return pltpu.async_copy(sup_hbm.at[sidx.at[i].at[j]], buf,
                                        sem)

            def scatter(j, buf, i=i):
                pltpu.sync_copy(buf, acc.at[didx.at[i].at[j]], add=True)

            @pl.when(nch > 0)
            def _():
                gather(0, rows_a, sem_a)

            def pair_body(p, carry):
                j0 = 2 * p
                j1 = j0 + 1

                @pl.when(j1 < nch)
                def _():
                    gather(j1, rows_b, sem_b)

                pltpu.make_async_copy(sup_hbm.at[sidx.at[0].at[0]], rows_a,
                                      sem_a).wait()
                scatter(j0, rows_a)

                @pl.when(j1 < nch)
                def _():
                    @pl.when(j1 + 1 < nch)
                    def _():
                        gather(j1 + 1, rows_a, sem_a)

                    pltpu.make_async_copy(sup_hbm.at[sidx.at[0].at[0]],
                                          rows_b, sem_b).wait()
                    scatter(j1, rows_b)

                return carry

            lax.fori_loop(0, (nch + 1) // 2, pair_body, 0)

        plsc.subcore_barrier()
        pltpu.sync_copy(acc.at[pl.ds(s * wb, wb)],
                        out_hbm.at[pl.ds(c * half + s * wb, wb)])

    return segsum


_BLK = 1000


def _mm_bias_body(x_ref, w_ref, b_ref, o_ref):
    o_ref[...] = (jnp.dot(x_ref[...], w_ref[...],
                          preferred_element_type=jnp.float32) + b_ref[...])


def _mm_bias(x2, w, b):
    n, k = x2.shape
    m = w.shape[1]
    return pl.pallas_call(
        _mm_bias_body,
        grid=(n // _BLK,),
        in_specs=[
            pl.BlockSpec((_BLK, k), lambda i: (i, 0)),
            pl.BlockSpec((k, m), lambda i: (0, 0)),
            pl.BlockSpec((1, m), lambda i: (0, 0)),
        ],
        out_specs=pl.BlockSpec((_BLK, m), lambda i: (i, 0)),
        out_shape=jax.ShapeDtypeStruct((n, m), jnp.float32),
    )(x2, w, b.reshape(1, m))


def _combine_body(agg_ref, res_ref, w2_ref, b2_ref, wn_ref, bn_ref,
                  out_ref, supn_ref):
    agg = agg_ref[...]
    r = (jnp.dot(res_ref[...], w2_ref[...],
                 preferred_element_type=jnp.float32) + b2_ref[...])
    g = jax.nn.sigmoid(agg + r)
    o = g * jnp.tanh(agg) + (1.0 - g) * r
    out_ref[...] = o
    supn_ref[...] = (jnp.dot(o, wn_ref[...],
                             preferred_element_type=jnp.float32) + bn_ref[...])


def _combine(agg, res, w2, b2, wn, bn):
    n, d = res.shape
    m = wn.shape[1]
    return pl.pallas_call(
        _combine_body,
        grid=(n // _BLK,),
        in_specs=[
            pl.BlockSpec((_BLK, d), lambda i: (i, 0)),
            pl.BlockSpec((_BLK, d), lambda i: (i, 0)),
            pl.BlockSpec((d, d), lambda i: (0, 0)),
            pl.BlockSpec((1, d), lambda i: (0, 0)),
            pl.BlockSpec((d, m), lambda i: (0, 0)),
            pl.BlockSpec((1, m), lambda i: (0, 0)),
        ],
        out_specs=[
            pl.BlockSpec((_BLK, d), lambda i: (i, 0)),
            pl.BlockSpec((_BLK, m), lambda i: (i, 0)),
        ],
        out_shape=[
            jax.ShapeDtypeStruct((n, d), jnp.float32),
            jax.ShapeDtypeStruct((n, m), jnp.float32),
        ],
    )(agg, res, w2, b2.reshape(1, d), wn, bn.reshape(1, m))


def _gru_body(gx_ref, h_ref, wh_ref, bh_ref, o_ref):
    d = h_ref.shape[1]
    gh = (jnp.dot(h_ref[...], wh_ref[...],
                  preferred_element_type=jnp.float32) + bh_ref[...])
    gx = gx_ref[...]
    i_r, i_i, i_n = gx[:, :d], gx[:, d:2 * d], gx[:, 2 * d:]
    h_r, h_i, h_n = gh[:, :d], gh[:, d:2 * d], gh[:, 2 * d:]
    rg = jax.nn.sigmoid(i_r + h_r)
    ig = jax.nn.sigmoid(i_i + h_i)
    ng = jnp.tanh(i_n + rg * h_n)
    o_ref[...] = ng + ig * (h_ref[...] - ng)


def _gru(gx, h, wh, bh):
    n, d = h.shape
    return pl.pallas_call(
        _gru_body,
        grid=(n // _BLK,),
        in_specs=[
            pl.BlockSpec((_BLK, 3 * d), lambda i: (i, 0)),
            pl.BlockSpec((_BLK, d), lambda i: (i, 0)),
            pl.BlockSpec((d, 3 * d), lambda i: (0, 0)),
            pl.BlockSpec((1, 3 * d), lambda i: (0, 0)),
        ],
        out_specs=pl.BlockSpec((_BLK, d), lambda i: (i, 0)),
        out_shape=jax.ShapeDtypeStruct((n, d), jnp.float32),
    )(gx, h, wh, bh.reshape(1, 3 * d))


def kernel(x, edge_index, W1, b1, W2, b2, Wx, bx, Wh, bh):
    t_steps, n_nodes, d = x.shape
    layers = W1.shape[1]
    n_edges = edge_index.shape[1]

    wb = _rup(-(-n_nodes // _NW), 8)
    out_rows = wb * _NW
    half = wb * _NS
    slabz = _rup(-(-(half + 8) // _NS), 8)
    acc_rows = slabz * _NS

    per_worker_unit = _CHUNK * _NW
    epad = _rup(n_edges, per_worker_unit)
    wch = epad // per_worker_unit

    pad = epad - n_edges
    srcp = jnp.concatenate(
        [edge_index[0], jnp.zeros((pad,), jnp.int32)]).reshape(_NW, wch, _CHUNK)
    dstp = jnp.concatenate(
        [edge_index[1], jnp.full((pad,), 2 * half, jnp.int32)]
    ).reshape(_NW, wch, _CHUNK)
    zero_blk = jnp.zeros((slabz, d), jnp.float32)

    part = _make_partition(wch, half)
    srcl, dstl, cnts = part(srcp, dstp)
    segsum = _make_segsum(d, wch, half, acc_rows, slabz, wb, out_rows)

    steps = t_steps * layers
    sup0 = jnp.stack([_mm_bias(x[t], W1[t, 0], b1[t, 0])
                      for t in range(t_steps)])

    w2s, b2s, wns, bns, sup0s, res0s, l0f, lastf = [], [], [], [], [], [], [], []
    znd = jnp.zeros((n_nodes, d), jnp.float32)
    for t in range(t_steps):
        for l in range(layers):
            w2s.append(W2[t, l])
            b2s.append(b2[t, l])
            if l + 1 < layers:
                wns.append(jnp.pad(W1[t, l + 1], ((0, 0), (0, 2 * d))))
                bns.append(jnp.pad(b1[t, l + 1], (0, 2 * d)))
            else:
                wns.append(Wx)
                bns.append(bx)
            sup0s.append(sup0[t] if l == 0 else znd)
            res0s.append(x[t] if l == 0 else znd)
            l0f.append(l == 0)
            lastf.append(l == layers - 1)
    xs = (jnp.stack(w2s), jnp.stack(b2s), jnp.stack(wns), jnp.stack(bns),
          jnp.stack(sup0s), jnp.stack(res0s),
          jnp.array(l0f), jnp.array(lastf))

    def step(carry, xso):
        res, supw, h = carry
        w2i, b2i, wni, bni, sup0i, res0i, is_l0, is_last = xso
        sup_in = jnp.where(is_l0, sup0i, supw[:, :d])
        res_in = jnp.where(is_l0, res0i, res)
        agg = segsum(sup_in, srcl, dstl, cnts, zero_blk)
        res_out, supw_out = _combine(agg, res_in, w2i, b2i, wni, bni)
        h_new = _gru(supw_out, h, Wh, bh)
        h_out = jnp.where(is_last, h_new, h)
        return (res_out, supw_out, h_out), h_out

    init = (znd, jnp.zeros((n_nodes, 3 * d), jnp.float32), znd)
    _, hs = lax.scan(step, init, xs)
    return hs[layers - 1::layers]
